# Initial kernel scaffold; baseline (speedup 1.0000x reference)
import jax
import jax.numpy as jnp
from jax.experimental import pallas as pl


def kernel(x, W_remap, b_remap, Wfc1, Wa1, Wfc2, Wa2, Wfc3, Wa3, Wfc4, Wa4):
    raise NotImplementedError("write your pallas kernel here")


# speedup vs baseline: 316.5637x; 316.5637x over previous
import jax
import jax.numpy as jnp
from jax.experimental import pallas as pl

B = 256
N = 100
NP = 128
IN_DIM = 16
INNER = 256
LATENT = 128
BB = 8


def _relu(v):
    return jnp.maximum(v, 0.0)


def _gat_block(h, w_ref, asrc_ref, adst_ref, residual):
    w = w_ref[...]
    asrc = asrc_ref[...]
    adst = adst_ref[...]
    z = jnp.dot(h, w, preferred_element_type=jnp.float32)
    col = jax.lax.broadcasted_iota(jnp.int32, (NP, NP), 1)
    valid = col < N
    outs = []
    for b in range(BB):
        zb = z[b * NP:(b + 1) * NP, :]
        ed = jax.lax.dot_general(zb, adst, (((1,), (0,)), ((), ())),
                                 preferred_element_type=jnp.float32)
        es = jax.lax.dot_general(asrc, zb, (((1,), (1,)), ((), ())),
                                 preferred_element_type=jnp.float32)
        e = ed + es
        e = jnp.where(e >= 0, e, 0.01 * e)
        e = jnp.where(valid, e, -1e30)
        m = jnp.max(e, axis=1, keepdims=True)
        p = jnp.exp(e - m)
        s = jnp.sum(p, axis=1, keepdims=True)
        a = p / s
        ob = jnp.dot(a, zb, preferred_element_type=jnp.float32)
        if residual:
            ob = _relu(ob + h[b * NP:(b + 1) * NP, :])
        outs.append(ob)
    return jnp.concatenate(outs, axis=0)


def _encoder_kernel(x_ref, wr_ref, b_ref,
                    w1_ref, as1_ref, ad1_ref,
                    w2_ref, as2_ref, ad2_ref,
                    w3_ref, as3_ref, ad3_ref,
                    w4_ref, as4_ref, ad4_ref,
                    out_ref):
    h = _relu(jnp.dot(x_ref[...], wr_ref[...],
                      preferred_element_type=jnp.float32) + b_ref[...])
    h = _gat_block(h, w1_ref, as1_ref, ad1_ref, True)
    h = _gat_block(h, w2_ref, as2_ref, ad2_ref, True)
    h = _gat_block(h, w3_ref, as3_ref, ad3_ref, True)
    out_ref[...] = _gat_block(h, w4_ref, as4_ref, ad4_ref, False)


def kernel(x, W_remap, b_remap, Wfc1, Wa1, Wfc2, Wa2, Wfc3, Wa3, Wfc4, Wa4):
    xp = jnp.pad(x, ((0, 0), (0, NP - N), (0, 0))).reshape(B * NP, IN_DIM)
    wr = W_remap.T
    br = b_remap.reshape(1, INNER)

    def split_a(Wa, dout):
        return Wa[:, :dout], Wa[:, dout:].T

    as1, ad1 = split_a(Wa1, INNER)
    as2, ad2 = split_a(Wa2, INNER)
    as3, ad3 = split_a(Wa3, INNER)
    as4, ad4 = split_a(Wa4, LATENT)

    full = lambda shp: pl.BlockSpec(shp, lambda i: (0, 0))
    out = pl.pallas_call(
        _encoder_kernel,
        grid=(B // BB,),
        in_specs=[
            pl.BlockSpec((BB * NP, IN_DIM), lambda i: (i, 0)),
            full((IN_DIM, INNER)), full((1, INNER)),
            full((INNER, INNER)), full((1, INNER)), full((INNER, 1)),
            full((INNER, INNER)), full((1, INNER)), full((INNER, 1)),
            full((INNER, INNER)), full((1, INNER)), full((INNER, 1)),
            full((INNER, LATENT)), full((1, LATENT)), full((LATENT, 1)),
        ],
        out_specs=pl.BlockSpec((BB * NP, LATENT), lambda i: (i, 0)),
        out_shape=jax.ShapeDtypeStruct((B * NP, LATENT), jnp.float32),
    )(xp, wr, br,
      Wfc1.T, as1, ad1,
      Wfc2.T, as2, ad2,
      Wfc3.T, as3, ad3,
      Wfc4.T, as4, ad4)
    return out.reshape(B, NP, LATENT)[:, :N, :]


# speedup vs baseline: 573.0561x; 1.8102x over previous
import jax
import jax.numpy as jnp
from jax.experimental import pallas as pl
from jax.experimental.pallas import tpu as pltpu

B = 256
N = 100
NP = 128
IN_DIM = 16
INNER = 256
LATENT = 128
BB = 8


def _relu(v):
    return jnp.maximum(v, 0.0)


def _gat_block(h, w_ref, asrc_ref, adst_ref, residual):
    w = w_ref[...]
    asrc = asrc_ref[...]
    adst = adst_ref[...]
    z = jnp.dot(h, w, preferred_element_type=jnp.float32)
    ed_all = jax.lax.dot_general(z, adst, (((1,), (0,)), ((), ())),
                                 preferred_element_type=jnp.float32)
    e_rows = []
    for b in range(BB):
        zb = z[b * NP:(b + 1) * NP, :]
        es = jax.lax.dot_general(asrc, zb, (((1,), (1,)), ((), ())),
                                 preferred_element_type=jnp.float32)
        e_rows.append(ed_all[b * NP:(b + 1) * NP, :] + es)
    e = jnp.concatenate(e_rows, axis=0)
    e = jnp.where(e >= 0, e, 0.01 * e)
    col = jax.lax.broadcasted_iota(jnp.int32, (BB * NP, NP), 1)
    e = jnp.where(col < N, e, -1e30)
    m = jnp.max(e, axis=1, keepdims=True)
    p = jnp.exp(e - m)
    s = jnp.sum(p, axis=1, keepdims=True)
    a = p * jax.lax.reciprocal(s)
    outs = []
    for b in range(BB):
        zb = z[b * NP:(b + 1) * NP, :]
        ab = a[b * NP:(b + 1) * NP, :]
        outs.append(jnp.dot(ab, zb, preferred_element_type=jnp.float32))
    out = jnp.concatenate(outs, axis=0)
    if residual:
        out = _relu(out + h)
    return out


def _encoder_kernel(x_ref, wr_ref, b_ref,
                    w1_ref, as1_ref, ad1_ref,
                    w2_ref, as2_ref, ad2_ref,
                    w3_ref, as3_ref, ad3_ref,
                    w4_ref, as4_ref, ad4_ref,
                    out_ref):
    h = _relu(jnp.dot(x_ref[...], wr_ref[...],
                      preferred_element_type=jnp.float32) + b_ref[...])
    h = _gat_block(h, w1_ref, as1_ref, ad1_ref, True)
    h = _gat_block(h, w2_ref, as2_ref, ad2_ref, True)
    h = _gat_block(h, w3_ref, as3_ref, ad3_ref, True)
    out_ref[...] = _gat_block(h, w4_ref, as4_ref, ad4_ref, False)


def kernel(x, W_remap, b_remap, Wfc1, Wa1, Wfc2, Wa2, Wfc3, Wa3, Wfc4, Wa4):
    xp = jnp.pad(x, ((0, 0), (0, NP - N), (0, 0))).reshape(B * NP, IN_DIM)
    wr = W_remap.T
    br = b_remap.reshape(1, INNER)

    def split_a(Wa, dout):
        return Wa[:, :dout], Wa[:, dout:].T

    as1, ad1 = split_a(Wa1, INNER)
    as2, ad2 = split_a(Wa2, INNER)
    as3, ad3 = split_a(Wa3, INNER)
    as4, ad4 = split_a(Wa4, LATENT)

    full = lambda shp: pl.BlockSpec(shp, lambda i: (0, 0))
    out = pl.pallas_call(
        _encoder_kernel,
        grid=(B // BB,),
        in_specs=[
            pl.BlockSpec((BB * NP, IN_DIM), lambda i: (i, 0)),
            full((IN_DIM, INNER)), full((1, INNER)),
            full((INNER, INNER)), full((1, INNER)), full((INNER, 1)),
            full((INNER, INNER)), full((1, INNER)), full((INNER, 1)),
            full((INNER, INNER)), full((1, INNER)), full((INNER, 1)),
            full((INNER, LATENT)), full((1, LATENT)), full((LATENT, 1)),
        ],
        out_specs=pl.BlockSpec((BB * NP, LATENT), lambda i: (i, 0)),
        out_shape=jax.ShapeDtypeStruct((B * NP, LATENT), jnp.float32),
        compiler_params=pltpu.CompilerParams(
            dimension_semantics=("parallel",)),
    )(xp, wr, br,
      Wfc1.T, as1, ad1,
      Wfc2.T, as2, ad2,
      Wfc3.T, as3, ad3,
      Wfc4.T, as4, ad4)
    return out.reshape(B, NP, LATENT)[:, :N, :]


# speedup vs baseline: 616.0543x; 1.0750x over previous
import jax
import jax.numpy as jnp
from jax.experimental import pallas as pl
from jax.experimental.pallas import tpu as pltpu

B = 256
N = 100
NP = 128
IN_DIM = 16
INNER = 256
LATENT = 128
BB = 8


def _relu(v):
    return jnp.maximum(v, 0.0)


def _gat_block(h, w_ref, asrc_ref, residual):
    wext = w_ref[...]
    dout = wext.shape[1] - 128
    asrc = asrc_ref[...]
    zext = jnp.dot(h, wext, preferred_element_type=jnp.float32)
    z = zext[:, :dout]
    ed_all = zext[:, dout:dout + 1]
    colrow = jax.lax.broadcasted_iota(jnp.int32, (1, NP), 1)
    e_rows = []
    for b in range(BB):
        zb = z[b * NP:(b + 1) * NP, :]
        es = jax.lax.dot_general(asrc, zb, (((1,), (1,)), ((), ())),
                                 preferred_element_type=jnp.float32)
        es = jnp.where(colrow < N, es, -1e30)
        e_rows.append(ed_all[b * NP:(b + 1) * NP, :] + es)
    e = jnp.concatenate(e_rows, axis=0)
    e = jnp.where(e >= 0, e, 0.01 * e)
    m = jnp.max(e, axis=1, keepdims=True)
    p = jnp.exp(e - m)
    s = jnp.sum(p, axis=1, keepdims=True)
    a = p * jax.lax.reciprocal(s)
    outs = []
    for b in range(BB):
        zb = z[b * NP:(b + 1) * NP, :]
        ab = a[b * NP:(b + 1) * NP, :]
        outs.append(jnp.dot(ab, zb, preferred_element_type=jnp.float32))
    out = jnp.concatenate(outs, axis=0)
    if residual:
        out = _relu(out + h)
    return out


def _encoder_kernel(x_ref, wr_ref, b_ref,
                    w1_ref, as1_ref,
                    w2_ref, as2_ref,
                    w3_ref, as3_ref,
                    w4_ref, as4_ref,
                    out_ref):
    h = _relu(jnp.dot(x_ref[...], wr_ref[...],
                      preferred_element_type=jnp.float32) + b_ref[...])
    h = _gat_block(h, w1_ref, as1_ref, True)
    h = _gat_block(h, w2_ref, as2_ref, True)
    h = _gat_block(h, w3_ref, as3_ref, True)
    out_ref[...] = _gat_block(h, w4_ref, as4_ref, False)


def kernel(x, W_remap, b_remap, Wfc1, Wa1, Wfc2, Wa2, Wfc3, Wa3, Wfc4, Wa4):
    xp = jnp.pad(x, ((0, 0), (0, NP - N), (0, 0))).reshape(B * NP, IN_DIM)
    wr = W_remap.T
    br = b_remap.reshape(1, INNER)

    def prep(Wfc, Wa, dout):
        adst_col = jnp.pad(Wfc.T @ Wa[:, dout:].T, ((0, 0), (0, 127)))
        return jnp.concatenate([Wfc.T, adst_col], axis=1), Wa[:, :dout]

    w1, as1 = prep(Wfc1, Wa1, INNER)
    w2, as2 = prep(Wfc2, Wa2, INNER)
    w3, as3 = prep(Wfc3, Wa3, INNER)
    w4, as4 = prep(Wfc4, Wa4, LATENT)

    full = lambda shp: pl.BlockSpec(shp, lambda i: (0, 0))
    out = pl.pallas_call(
        _encoder_kernel,
        grid=(B // BB,),
        in_specs=[
            pl.BlockSpec((BB * NP, IN_DIM), lambda i: (i, 0)),
            full((IN_DIM, INNER)), full((1, INNER)),
            full((INNER, INNER + 128)), full((1, INNER)),
            full((INNER, INNER + 128)), full((1, INNER)),
            full((INNER, INNER + 128)), full((1, INNER)),
            full((INNER, LATENT + 128)), full((1, LATENT)),
        ],
        out_specs=pl.BlockSpec((BB * NP, LATENT), lambda i: (i, 0)),
        out_shape=jax.ShapeDtypeStruct((B * NP, LATENT), jnp.float32),
        compiler_params=pltpu.CompilerParams(
            dimension_semantics=("parallel",)),
    )(xp, wr, br, w1, as1, w2, as2, w3, as3, w4, as4)
    return out.reshape(B, NP, LATENT)[:, :N, :]


# speedup vs baseline: 894.9173x; 1.4527x over previous
import jax
import jax.numpy as jnp
from jax.experimental import pallas as pl
from jax.experimental.pallas import tpu as pltpu

B = 256
N = 100
NP = 128
IN_DIM = 16
INNER = 256
LATENT = 128
BB = 16


def _relu(v):
    return jnp.maximum(v, 0.0)


def _gat_block(h, w_ref, asrc_ref, residual):
    wext = w_ref[...]
    dout = wext.shape[1] - 128
    asrc = asrc_ref[...]
    zext = jnp.dot(h, wext, preferred_element_type=jnp.float32)
    z = zext[:, :dout]
    ed_all = zext[:, dout:dout + 1]
    colrow = jax.lax.broadcasted_iota(jnp.int32, (1, NP), 1)
    e_rows = []
    for b in range(BB):
        zb = z[b * NP:(b + 1) * NP, :]
        es = jax.lax.dot_general(asrc, zb, (((1,), (1,)), ((), ())),
                                 preferred_element_type=jnp.float32)
        es = jnp.where(colrow < N, es, -1e30)
        e_rows.append(ed_all[b * NP:(b + 1) * NP, :] + es)
    e = jnp.concatenate(e_rows, axis=0)
    e = jnp.where(e >= 0, e, 0.01 * e)
    m = jnp.max(e, axis=1, keepdims=True)
    p = jnp.exp(e - m)
    s = jnp.sum(p, axis=1, keepdims=True)
    a = p * jax.lax.reciprocal(s)
    outs = []
    for b in range(BB):
        zb = z[b * NP:(b + 1) * NP, :]
        ab = a[b * NP:(b + 1) * NP, :]
        outs.append(jnp.dot(ab, zb, preferred_element_type=jnp.float32))
    out = jnp.concatenate(outs, axis=0)
    if residual:
        out = _relu(out + h)
    return out


def _encoder_kernel(x_ref, wr_ref, b_ref,
                    w1_ref, as1_ref,
                    w2_ref, as2_ref,
                    w3_ref, as3_ref,
                    w4_ref, as4_ref,
                    out_ref):
    h = _relu(jnp.dot(x_ref[...], wr_ref[...],
                      preferred_element_type=jnp.float32) + b_ref[...])
    h = _gat_block(h, w1_ref, as1_ref, True)
    h = _gat_block(h, w2_ref, as2_ref, True)
    h = _gat_block(h, w3_ref, as3_ref, True)
    out = _gat_block(h, w4_ref, as4_ref, False)
    for b in range(BB):
        out_ref[b, :, :] = out[b * NP:b * NP + N, :]


def kernel(x, W_remap, b_remap, Wfc1, Wa1, Wfc2, Wa2, Wfc3, Wa3, Wfc4, Wa4):
    xp = jnp.pad(x, ((0, 0), (0, NP - N), (0, 0))).reshape(B * NP, IN_DIM)
    wr = W_remap.T
    br = b_remap.reshape(1, INNER)

    def prep(Wfc, Wa, dout):
        adst_col = jnp.pad(Wfc.T @ Wa[:, dout:].T, ((0, 0), (0, 127)))
        return jnp.concatenate([Wfc.T, adst_col], axis=1), Wa[:, :dout]

    w1, as1 = prep(Wfc1, Wa1, INNER)
    w2, as2 = prep(Wfc2, Wa2, INNER)
    w3, as3 = prep(Wfc3, Wa3, INNER)
    w4, as4 = prep(Wfc4, Wa4, LATENT)

    full = lambda shp: pl.BlockSpec(shp, lambda i: (0, 0))
    out = pl.pallas_call(
        _encoder_kernel,
        grid=(B // BB,),
        in_specs=[
            pl.BlockSpec((BB * NP, IN_DIM), lambda i: (i, 0)),
            full((IN_DIM, INNER)), full((1, INNER)),
            full((INNER, INNER + 128)), full((1, INNER)),
            full((INNER, INNER + 128)), full((1, INNER)),
            full((INNER, INNER + 128)), full((1, INNER)),
            full((INNER, LATENT + 128)), full((1, LATENT)),
        ],
        out_specs=pl.BlockSpec((BB, N, LATENT), lambda i: (i, 0, 0)),
        out_shape=jax.ShapeDtypeStruct((B, N, LATENT), jnp.float32),
        compiler_params=pltpu.CompilerParams(
            dimension_semantics=("parallel",)),
    )(xp, wr, br, w1, as1, w2, as2, w3, as3, w4, as4)
    return out


# speedup vs baseline: 924.0890x; 1.0326x over previous
import jax
import jax.numpy as jnp
from jax.experimental import pallas as pl
from jax.experimental.pallas import tpu as pltpu

B = 256
N = 100
NP = 128
IN_DIM = 16
INNER = 256
LATENT = 128
BB = 32


def _relu(v):
    return jnp.maximum(v, 0.0)


def _gat_block(h, w_ref, asrc_ref, residual):
    wext = w_ref[...]
    dout = wext.shape[1] - 128
    asrc = asrc_ref[...]
    zext = jnp.dot(h, wext, preferred_element_type=jnp.float32)
    z = zext[:, :dout]
    ed_all = zext[:, dout:dout + 1]
    colrow = jax.lax.broadcasted_iota(jnp.int32, (1, NP), 1)
    e_rows = []
    for b in range(BB):
        zb = z[b * NP:(b + 1) * NP, :]
        es = jax.lax.dot_general(asrc, zb, (((1,), (1,)), ((), ())),
                                 preferred_element_type=jnp.float32)
        es = jnp.where(colrow < N, es, -1e30)
        e_rows.append(ed_all[b * NP:(b + 1) * NP, :] + es)
    e = jnp.concatenate(e_rows, axis=0)
    e = jnp.where(e >= 0, e, 0.01 * e)
    m = jnp.max(e, axis=1, keepdims=True)
    p = jnp.exp(e - m)
    s = jnp.sum(p, axis=1, keepdims=True)
    a = p * jax.lax.reciprocal(s)
    outs = []
    for b in range(BB):
        zb = z[b * NP:(b + 1) * NP, :]
        ab = a[b * NP:(b + 1) * NP, :]
        outs.append(jnp.dot(ab, zb, preferred_element_type=jnp.float32))
    out = jnp.concatenate(outs, axis=0)
    if residual:
        out = _relu(out + h)
    return out


def _encoder_kernel(x_ref, wr_ref, b_ref,
                    w1_ref, as1_ref,
                    w2_ref, as2_ref,
                    w3_ref, as3_ref,
                    w4_ref, as4_ref,
                    out_ref):
    h = _relu(jnp.dot(x_ref[...], wr_ref[...],
                      preferred_element_type=jnp.float32) + b_ref[...])
    h = _gat_block(h, w1_ref, as1_ref, True)
    h = _gat_block(h, w2_ref, as2_ref, True)
    h = _gat_block(h, w3_ref, as3_ref, True)
    out = _gat_block(h, w4_ref, as4_ref, False)
    for b in range(BB):
        out_ref[b, :, :] = out[b * NP:b * NP + N, :]


def kernel(x, W_remap, b_remap, Wfc1, Wa1, Wfc2, Wa2, Wfc3, Wa3, Wfc4, Wa4):
    xp = jnp.pad(x, ((0, 0), (0, NP - N), (0, 0))).reshape(B * NP, IN_DIM)
    wr = W_remap.T
    br = b_remap.reshape(1, INNER)

    def prep(Wfc, Wa, dout):
        adst_col = jnp.pad(Wfc.T @ Wa[:, dout:].T, ((0, 0), (0, 127)))
        return jnp.concatenate([Wfc.T, adst_col], axis=1), Wa[:, :dout]

    w1, as1 = prep(Wfc1, Wa1, INNER)
    w2, as2 = prep(Wfc2, Wa2, INNER)
    w3, as3 = prep(Wfc3, Wa3, INNER)
    w4, as4 = prep(Wfc4, Wa4, LATENT)

    full = lambda shp: pl.BlockSpec(shp, lambda i: (0, 0))
    out = pl.pallas_call(
        _encoder_kernel,
        grid=(B // BB,),
        in_specs=[
            pl.BlockSpec((BB * NP, IN_DIM), lambda i: (i, 0)),
            full((IN_DIM, INNER)), full((1, INNER)),
            full((INNER, INNER + 128)), full((1, INNER)),
            full((INNER, INNER + 128)), full((1, INNER)),
            full((INNER, INNER + 128)), full((1, INNER)),
            full((INNER, LATENT + 128)), full((1, LATENT)),
        ],
        out_specs=pl.BlockSpec((BB, N, LATENT), lambda i: (i, 0, 0)),
        out_shape=jax.ShapeDtypeStruct((B, N, LATENT), jnp.float32),
        compiler_params=pltpu.CompilerParams(
            dimension_semantics=("parallel",)),
    )(xp, wr, br, w1, as1, w2, as2, w3, as3, w4, as4)
    return out
